# baseline (device time: 40309 ns/iter reference)
import jax
import jax.numpy as jnp
from jax import lax
from jax.experimental import pallas as pl
from jax.experimental.pallas import tpu as pltpu

MESH = pl.DeviceIdType.MESH


def kernel(Q, K, V):
    b, s, h, d = Q.shape
    bh = b * h
    half = bh // 2
    scale = d ** -0.5

    def body(q_ref, k_ref, v_ref, o_ref, ql, kl, vl, kr, vr, ssem, rsem):
        my_x = lax.axis_index("x")
        my_y = lax.axis_index("y")
        my_z = lax.axis_index("z")
        z_peer = (my_x, my_y, 1 - my_z)
        x_peer = (1 - my_x, my_y, my_z)
        y_peer = (my_x, 1 - my_y, my_z)
        is_kk = my_x == my_y

        for i in range(bh):
            bi, hi = divmod(i, h)
            ql[i] = (q_ref[bi, :, hi, :] * scale).astype(jnp.bfloat16)
            kl[i] = k_ref[bi, :, hi, :].astype(jnp.bfloat16)
            vl[i] = v_ref[bi, :, hi, :].astype(jnp.bfloat16)

        bar = pltpu.get_barrier_semaphore()
        for p in (z_peer, x_peer, y_peer):
            pl.semaphore_signal(bar, inc=1, device_id=p, device_id_type=MESH)
        pl.semaphore_wait(bar, 3)

        lo = pl.ds(0, half)
        hi_ = pl.ds(half, half)

        @pl.when(is_kk)
        def _():
            for c, sl in enumerate((lo, hi_)):
                pltpu.make_async_remote_copy(
                    src_ref=kl.at[sl], dst_ref=kr.at[sl],
                    send_sem=ssem.at[c], recv_sem=rsem.at[c],
                    device_id=z_peer, device_id_type=MESH,
                ).start()

        @pl.when(jnp.logical_not(is_kk))
        def _():
            for c, sl in enumerate((lo, hi_)):
                pltpu.make_async_remote_copy(
                    src_ref=vl.at[sl], dst_ref=vr.at[sl],
                    send_sem=ssem.at[c], recv_sem=rsem.at[c],
                    device_id=z_peer, device_id_type=MESH,
                ).start()

        dn_qk = (((2,), (2,)), ((0,), (0,)))
        dn_pv = (((2,), (1,)), ((0,), (0,)))

        qv = ql[...]
        s0 = lax.dot_general(qv, kl[...], dn_qk,
                             preferred_element_type=jnp.float32)
        p0 = jnp.exp(s0)
        l0 = jnp.sum(p0, axis=2)
        o0 = lax.dot_general(p0.astype(jnp.bfloat16), vl[...], dn_pv,
                             preferred_element_type=jnp.float32)

        def wait_a(c, sl):
            pltpu.make_async_remote_copy(
                src_ref=kl.at[sl], dst_ref=kr.at[sl],
                send_sem=ssem.at[c], recv_sem=rsem.at[c],
                device_id=z_peer, device_id_type=MESH,
            ).wait()

        wait_a(0, lo)

        @pl.when(is_kk)
        def _():
            pltpu.make_async_remote_copy(
                src_ref=kr.at[lo], dst_ref=kr.at[lo],
                send_sem=ssem.at[2], recv_sem=rsem.at[2],
                device_id=x_peer, device_id_type=MESH,
            ).start()

        @pl.when(jnp.logical_not(is_kk))
        def _():
            pltpu.make_async_remote_copy(
                src_ref=vr.at[lo], dst_ref=vr.at[lo],
                send_sem=ssem.at[2], recv_sem=rsem.at[2],
                device_id=x_peer, device_id_type=MESH,
            ).start()

        wait_a(1, hi_)

        @pl.when(is_kk)
        def _():
            pltpu.make_async_remote_copy(
                src_ref=kr.at[hi_], dst_ref=kr.at[hi_],
                send_sem=ssem.at[3], recv_sem=rsem.at[3],
                device_id=y_peer, device_id_type=MESH,
            ).start()

        @pl.when(jnp.logical_not(is_kk))
        def _():
            pltpu.make_async_remote_copy(
                src_ref=vr.at[hi_], dst_ref=vr.at[hi_],
                send_sem=ssem.at[3], recv_sem=rsem.at[3],
                device_id=y_peer, device_id_type=MESH,
            ).start()

        for c, sl in ((2, lo), (3, hi_)):
            pltpu.make_async_remote_copy(
                src_ref=kr.at[sl], dst_ref=kr.at[sl],
                send_sem=ssem.at[c], recv_sem=rsem.at[c],
                device_id=x_peer, device_id_type=MESH,
            ).wait()

        s1 = lax.dot_general(qv, kr[...], dn_qk,
                             preferred_element_type=jnp.float32)
        p1 = jnp.exp(s1)
        l1 = jnp.sum(p1, axis=2)
        o1 = lax.dot_general(p1.astype(jnp.bfloat16), vr[...], dn_pv,
                             preferred_element_type=jnp.float32)

        o = (o0 + o1) * (1.0 / (l0 + l1))[:, :, None]
        for i in range(bh):
            bi, hi2 = divmod(i, h)
            o_ref[bi, :, hi2, :] = o[i]

    return pl.pallas_call(
        body,
        out_shape=jax.ShapeDtypeStruct((b, s, h, d), jnp.float32),
        in_specs=[pl.BlockSpec(memory_space=pltpu.VMEM)] * 3,
        out_specs=pl.BlockSpec(memory_space=pltpu.VMEM),
        scratch_shapes=[
            pltpu.VMEM((bh, s, d), jnp.bfloat16),
            pltpu.VMEM((bh, s, d), jnp.bfloat16),
            pltpu.VMEM((bh, s, d), jnp.bfloat16),
            pltpu.VMEM((bh, s, d), jnp.bfloat16),
            pltpu.VMEM((bh, s, d), jnp.bfloat16),
            pltpu.SemaphoreType.DMA((4,)),
            pltpu.SemaphoreType.DMA((4,)),
        ],
        compiler_params=pltpu.CompilerParams(collective_id=0),
    )(Q, K, V)


# device time: 28268 ns/iter; 1.4260x vs baseline; 1.4260x over previous
import jax
import jax.numpy as jnp
from jax import lax
from jax.experimental import pallas as pl
from jax.experimental.pallas import tpu as pltpu

MESH = pl.DeviceIdType.MESH
N_CHUNK = 4


def kernel(Q, K, V):
    b, s, h, d = Q.shape
    bh = b * h
    csz = bh // N_CHUNK
    scale = d ** -0.5

    def to_heads(x):
        return jnp.reshape(jnp.transpose(x, (0, 2, 1, 3)), (bh, s, d)).astype(
            jnp.bfloat16
        )

    Qb = to_heads(Q * scale)
    Kb = to_heads(K)
    Vb = to_heads(V)

    def body(q_ref, k_ref, v_ref, o_ref, kr, vr, ssem, rsem):
        my_x = lax.axis_index("x")
        my_y = lax.axis_index("y")
        my_z = lax.axis_index("z")
        z_peer = (my_x, my_y, 1 - my_z)
        x_peer = (1 - my_x, my_y, my_z)
        y_peer = (my_x, 1 - my_y, my_z)
        is_kk = my_x == my_y

        sl = [pl.ds(c * csz, csz) for c in range(N_CHUNK)]
        fwd_peer = [x_peer, x_peer, y_peer, y_peer]

        bar = pltpu.get_barrier_semaphore()
        for p in (z_peer, x_peer, y_peer):
            pl.semaphore_signal(bar, inc=1, device_id=p, device_id_type=MESH)
        pl.semaphore_wait(bar, 3)

        @pl.when(is_kk)
        def _():
            for c in range(N_CHUNK):
                pltpu.make_async_remote_copy(
                    src_ref=k_ref.at[sl[c]], dst_ref=kr.at[sl[c]],
                    send_sem=ssem.at[c], recv_sem=rsem.at[c],
                    device_id=z_peer, device_id_type=MESH,
                ).start()

        @pl.when(jnp.logical_not(is_kk))
        def _():
            for c in range(N_CHUNK):
                pltpu.make_async_remote_copy(
                    src_ref=v_ref.at[sl[c]], dst_ref=vr.at[sl[c]],
                    send_sem=ssem.at[c], recv_sem=rsem.at[c],
                    device_id=z_peer, device_id_type=MESH,
                ).start()

        dn_qk = (((2,), (2,)), ((0,), (0,)))
        dn_pv = (((2,), (1,)), ((0,), (0,)))

        def attn_block(qv, kv, vv):
            st = lax.dot_general(qv, kv, dn_qk,
                                 preferred_element_type=jnp.float32)
            pt = jnp.exp(st)
            lt = jnp.sum(pt, axis=2)
            ot = lax.dot_general(pt.astype(jnp.bfloat16), vv, dn_pv,
                                 preferred_element_type=jnp.float32)
            return lt, ot

        qv = q_ref[...]
        l0, o0 = attn_block(qv, k_ref[...], v_ref[...])

        def wait_pair(c, buf_a, buf_b):
            pltpu.make_async_remote_copy(
                src_ref=buf_a.at[sl[c]], dst_ref=buf_b.at[sl[c]],
                send_sem=ssem.at[c], recv_sem=rsem.at[c],
                device_id=z_peer, device_id_type=MESH,
            ).wait()

        for c in range(N_CHUNK):
            wait_pair(c, k_ref, kr)

            @pl.when(is_kk)
            def _(c=c):
                pltpu.make_async_remote_copy(
                    src_ref=kr.at[sl[c]], dst_ref=kr.at[sl[c]],
                    send_sem=ssem.at[N_CHUNK + c], recv_sem=rsem.at[N_CHUNK + c],
                    device_id=fwd_peer[c], device_id_type=MESH,
                ).start()

            @pl.when(jnp.logical_not(is_kk))
            def _(c=c):
                pltpu.make_async_remote_copy(
                    src_ref=vr.at[sl[c]], dst_ref=vr.at[sl[c]],
                    send_sem=ssem.at[N_CHUNK + c], recv_sem=rsem.at[N_CHUNK + c],
                    device_id=fwd_peer[c], device_id_type=MESH,
                ).start()

        def wait_fwd(c):
            pltpu.make_async_remote_copy(
                src_ref=kr.at[sl[c]], dst_ref=kr.at[sl[c]],
                send_sem=ssem.at[N_CHUNK + c], recv_sem=rsem.at[N_CHUNK + c],
                device_id=fwd_peer[c], device_id_type=MESH,
            ).wait()

        half = pl.ds(0, bh // 2)
        wait_fwd(0)
        wait_fwd(1)
        l1a, o1a = attn_block(qv[: bh // 2], kr[half], vr[half])

        wait_fwd(2)
        wait_fwd(3)
        hhalf = pl.ds(bh // 2, bh // 2)
        l1b, o1b = attn_block(qv[bh // 2:], kr[hhalf], vr[hhalf])

        l1 = jnp.concatenate([l1a, l1b], axis=0)
        o1 = jnp.concatenate([o1a, o1b], axis=0)
        o_ref[...] = (o0 + o1) * (1.0 / (l0 + l1))[:, :, None]

    out = pl.pallas_call(
        body,
        out_shape=jax.ShapeDtypeStruct((bh, s, d), jnp.float32),
        in_specs=[pl.BlockSpec(memory_space=pltpu.VMEM)] * 3,
        out_specs=pl.BlockSpec(memory_space=pltpu.VMEM),
        scratch_shapes=[
            pltpu.VMEM((bh, s, d), jnp.bfloat16),
            pltpu.VMEM((bh, s, d), jnp.bfloat16),
            pltpu.SemaphoreType.DMA((2 * N_CHUNK,)),
            pltpu.SemaphoreType.DMA((2 * N_CHUNK,)),
        ],
        compiler_params=pltpu.CompilerParams(collective_id=0),
    )(Qb, Kb, Vb)

    return jnp.transpose(jnp.reshape(out, (b, h, s, d)), (0, 2, 1, 3))


# device time: 23766 ns/iter; 1.6961x vs baseline; 1.1894x over previous
import jax
import jax.numpy as jnp
from jax import lax
from jax.experimental import pallas as pl
from jax.experimental.pallas import tpu as pltpu

MESH = pl.DeviceIdType.MESH
N_CHUNK = 4


def kernel(Q, K, V):
    b, s, h, d = Q.shape
    bh = b * h
    csz = bh // N_CHUNK
    scale = d ** -0.5

    def to_heads(x):
        return jnp.reshape(jnp.transpose(x, (0, 2, 1, 3)), (bh, s, d)).astype(
            jnp.bfloat16
        )

    Qb = to_heads(Q * scale)
    Kb = to_heads(K)
    Vb = to_heads(V)

    def body(q_ref, k_ref, v_ref, o_ref, kr, vr, ssem, rsem):
        my_x = lax.axis_index("x")
        my_y = lax.axis_index("y")
        my_z = lax.axis_index("z")
        z_peer = (my_x, my_y, 1 - my_z)
        x_peer = (1 - my_x, my_y, my_z)
        y_peer = (my_x, 1 - my_y, my_z)
        is_kk = my_x == my_y
        fwd_lo = my_x == 0

        sl = [pl.ds(c * csz, csz) for c in range(N_CHUNK)]

        bar = pltpu.get_barrier_semaphore()
        for p in (z_peer, x_peer, y_peer):
            pl.semaphore_signal(bar, inc=1, device_id=p, device_id_type=MESH)
        pl.semaphore_wait(bar, 3)

        def start_z(src, dst, order):
            for p, cid in enumerate(order):
                pltpu.make_async_remote_copy(
                    src_ref=src.at[sl[cid]], dst_ref=dst.at[sl[cid]],
                    send_sem=ssem.at[p], recv_sem=rsem.at[p],
                    device_id=z_peer, device_id_type=MESH,
                ).start()

        for kk in (True, False):
            for flo in (True, False):
                @pl.when((is_kk == kk) & (fwd_lo == flo))
                def _(kk=kk, flo=flo):
                    src, dst = (k_ref, kr) if kk else (v_ref, vr)
                    start_z(src, dst, (0, 1, 2, 3) if flo else (2, 3, 0, 1))

        dn_qk = (((2,), (2,)), ((0,), (0,)))
        dn_pv = (((2,), (1,)), ((0,), (0,)))

        def attn_block(qv, kv, vv):
            st = lax.dot_general(qv, kv, dn_qk,
                                 preferred_element_type=jnp.float32)
            pt = jnp.exp(st)
            lt = jnp.sum(pt, axis=2)
            ot = lax.dot_general(pt.astype(jnp.bfloat16), vv, dn_pv,
                                 preferred_element_type=jnp.float32)
            return lt, ot

        qv = q_ref[...]
        l0, o0 = attn_block(qv, k_ref[...], v_ref[...])

        def wait_z(p):
            pltpu.make_async_remote_copy(
                src_ref=kr.at[sl[p]], dst_ref=kr.at[sl[p]],
                send_sem=ssem.at[p], recv_sem=rsem.at[p],
                device_id=z_peer, device_id_type=MESH,
            ).wait()

        for p in range(2):
            wait_z(p)
            for kk in (True, False):
                for flo in (True, False):
                    @pl.when((is_kk == kk) & (fwd_lo == flo))
                    def _(kk=kk, flo=flo, p=p):
                        buf = kr if kk else vr
                        cid = (0, 1)[p] if flo else (2, 3)[p]
                        for t, peer in enumerate((x_peer, y_peer)):
                            pltpu.make_async_remote_copy(
                                src_ref=buf.at[sl[cid]],
                                dst_ref=buf.at[sl[cid]],
                                send_sem=ssem.at[4 + 2 * p + t],
                                recv_sem=rsem.at[4 + cid],
                                device_id=peer, device_id_type=MESH,
                            ).start()

        wait_z(2)
        wait_z(3)

        for c in range(N_CHUNK):
            pltpu.make_async_remote_copy(
                src_ref=vr.at[sl[c]], dst_ref=vr.at[sl[c]],
                send_sem=ssem.at[0], recv_sem=rsem.at[4 + c],
                device_id=x_peer, device_id_type=MESH,
            ).wait_recv()
        for t in range(4):
            pltpu.make_async_remote_copy(
                src_ref=vr.at[sl[0]], dst_ref=vr.at[sl[0]],
                send_sem=ssem.at[4 + t], recv_sem=rsem.at[0],
                device_id=x_peer, device_id_type=MESH,
            ).wait_send()

        l1, o1 = attn_block(qv, kr[...], vr[...])
        o_ref[...] = (o0 + o1) * (1.0 / (l0 + l1))[:, :, None]

    out = pl.pallas_call(
        body,
        out_shape=jax.ShapeDtypeStruct((bh, s, d), jnp.float32),
        in_specs=[pl.BlockSpec(memory_space=pltpu.VMEM)] * 3,
        out_specs=pl.BlockSpec(memory_space=pltpu.VMEM),
        scratch_shapes=[
            pltpu.VMEM((bh, s, d), jnp.bfloat16),
            pltpu.VMEM((bh, s, d), jnp.bfloat16),
            pltpu.SemaphoreType.DMA((8,)),
            pltpu.SemaphoreType.DMA((8,)),
        ],
        compiler_params=pltpu.CompilerParams(collective_id=0),
    )(Qb, Kb, Vb)

    return jnp.transpose(jnp.reshape(out, (b, h, s, d)), (0, 2, 1, 3))
